# baseline (speedup 1.0000x reference)
import functools

import jax
import jax.numpy as jnp
from jax import lax
from jax.experimental import pallas as pl
from jax.experimental.pallas import tpu as pltpu
from jax.experimental.pallas import tpu_sc as plsc

HIDDEN = 1024
NUM_LAYERS = 24
NUM_EXPERTS = 8
NC = 2
NS = 16
NW = NC * NS

CHUNK = 8
NBUF = 12


@functools.lru_cache(maxsize=None)
def _make_sc_kernel(B: int):
    BPW = B // NW
    NCHUNK = BPW // CHUNK

    mesh = plsc.VectorSubcoreMesh(core_axis_name="c", subcore_axis_name="s")

    @functools.partial(
        pl.kernel,
        mesh=mesh,
        out_type=(
            jax.ShapeDtypeStruct((B, HIDDEN), jnp.float32),
            jax.ShapeDtypeStruct((B,), jnp.int32),
        ),
        scratch_types=(
            [pltpu.VMEM((BPW,), jnp.int32),
             pltpu.VMEM((NBUF, CHUNK, HIDDEN), jnp.float32)]
            + [pltpu.SemaphoreType.DMA] * (2 * NBUF)
        ),
    )
    def k(ids_hbm, table_hbm, emb_out, pos_out, idx_v, rows_v, *sems):
        wid = lax.axis_index("s") * NC + lax.axis_index("c")
        base = wid * BPW

        pltpu.sync_copy(ids_hbm.at[pl.ds(base, BPW)], idx_v)

        gsems = sems[:NBUF]
        osems = sems[NBUF:]

        def start_gather(c):
            return pltpu.async_copy(
                table_hbm.at[idx_v.at[pl.ds(c * CHUNK, CHUNK)]],
                rows_v.at[c % NBUF], gsems[c % NBUF])

        gathers = [None] * NCHUNK
        ocopies = [None] * NCHUNK
        for c in range(min(NBUF - 1, NCHUNK)):
            gathers[c] = start_gather(c)
        for c in range(NCHUNK):
            gathers[c].wait()
            ocopies[c] = pltpu.async_copy(
                rows_v.at[c % NBUF],
                emb_out.at[pl.ds(base + c * CHUNK, CHUNK)], osems[c % NBUF])
            nxt = c + NBUF - 1
            if nxt < NCHUNK:
                if c >= 1:
                    ocopies[c - 1].wait()
                gathers[nxt] = start_gather(nxt)
        for c in range(max(0, NCHUNK - NBUF), NCHUNK):
            ocopies[c].wait()

        seq_len = B // 4
        pbase = base % seq_len
        for j in range(BPW // 16):
            idx_v[pl.ds(j * 16, 16)] = (
                pbase + j * 16 + lax.iota(jnp.int32, 16))
        pltpu.sync_copy(idx_v, pos_out.at[pl.ds(base, BPW)])

    return k


def _zeros_body(out_ref):
    out_ref[...] = jnp.zeros_like(out_ref)


@functools.lru_cache(maxsize=None)
def _make_gate_zeros(bsz: int, seq_len: int):
    return pl.pallas_call(
        _zeros_body,
        out_shape=jax.ShapeDtypeStruct(
            (NUM_LAYERS, bsz, NUM_EXPERTS, seq_len), jnp.float32),
        grid=(NUM_LAYERS,),
        out_specs=pl.BlockSpec(
            (1, bsz, NUM_EXPERTS, seq_len), lambda i: (i, 0, 0, 0)),
    )


def kernel(input_ids, embed_weight):
    bsz, seq_len = input_ids.shape
    B = bsz * seq_len
    emb, pos = _make_sc_kernel(B)(input_ids.reshape(B), embed_weight)
    gate_t = _make_gate_zeros(bsz, seq_len)()
    return (emb.reshape(bsz, seq_len, HIDDEN),
            pos.reshape(bsz, seq_len),
            jnp.transpose(gate_t, (0, 1, 3, 2)))


# speedup vs baseline: 1.0488x; 1.0488x over previous
import functools

import jax
import jax.numpy as jnp
from jax import lax
from jax.experimental import pallas as pl
from jax.experimental.pallas import tpu as pltpu
from jax.experimental.pallas import tpu_sc as plsc

HIDDEN = 1024
NUM_LAYERS = 24
NUM_EXPERTS = 8
NC = 2
NS = 16
NW = NC * NS

CHUNK = 16
NBUF = 6


@functools.lru_cache(maxsize=None)
def _make_sc_kernel(B: int):
    BPW = B // NW
    NCHUNK = BPW // CHUNK

    mesh = plsc.VectorSubcoreMesh(core_axis_name="c", subcore_axis_name="s")

    @functools.partial(
        pl.kernel,
        mesh=mesh,
        out_type=jax.ShapeDtypeStruct((B, HIDDEN), jnp.float32),
        scratch_types=(
            [pltpu.VMEM((BPW,), jnp.int32),
             pltpu.VMEM((NBUF, CHUNK, HIDDEN), jnp.float32)]
            + [pltpu.SemaphoreType.DMA] * (2 * NBUF)
        ),
    )
    def k(ids_hbm, table_hbm, emb_out, idx_v, rows_v, *sems):
        wid = lax.axis_index("s") * NC + lax.axis_index("c")
        base = wid * BPW

        pltpu.sync_copy(ids_hbm.at[pl.ds(base, BPW)], idx_v)

        gsems = sems[:NBUF]
        osems = sems[NBUF:]

        def start_gather(c):
            return pltpu.async_copy(
                table_hbm.at[idx_v.at[pl.ds(c * CHUNK, CHUNK)]],
                rows_v.at[c % NBUF], gsems[c % NBUF])

        gathers = [None] * NCHUNK
        ocopies = [None] * NCHUNK
        for c in range(min(NBUF - 1, NCHUNK)):
            gathers[c] = start_gather(c)
        for c in range(NCHUNK):
            gathers[c].wait()
            ocopies[c] = pltpu.async_copy(
                rows_v.at[c % NBUF],
                emb_out.at[pl.ds(base + c * CHUNK, CHUNK)], osems[c % NBUF])
            nxt = c + NBUF - 1
            if nxt < NCHUNK:
                if c >= 1:
                    ocopies[c - 1].wait()
                gathers[nxt] = start_gather(nxt)
        for c in range(max(0, NCHUNK - NBUF), NCHUNK):
            ocopies[c].wait()

    return k


def _aux_body(gate_ref, pos_ref):
    gate_ref[...] = jnp.zeros_like(gate_ref)

    @pl.when(pl.program_id(0) == 0)
    def _():
        pos_ref[...] = jax.lax.broadcasted_iota(
            jnp.int32, pos_ref.shape, dimension=1)


@functools.lru_cache(maxsize=None)
def _make_aux(bsz: int, seq_len: int):
    return pl.pallas_call(
        _aux_body,
        out_shape=(
            jax.ShapeDtypeStruct(
                (NUM_LAYERS, bsz, NUM_EXPERTS, seq_len), jnp.float32),
            jax.ShapeDtypeStruct((bsz, seq_len), jnp.int32),
        ),
        grid=(NUM_LAYERS,),
        out_specs=(
            pl.BlockSpec(
                (1, bsz, NUM_EXPERTS, seq_len), lambda i: (i, 0, 0, 0)),
            pl.BlockSpec((bsz, seq_len), lambda i: (0, 0)),
        ),
    )


def kernel(input_ids, embed_weight):
    bsz, seq_len = input_ids.shape
    B = bsz * seq_len
    emb = _make_sc_kernel(B)(input_ids.reshape(B), embed_weight)
    gate_t, pos = _make_aux(bsz, seq_len)()
    return (emb.reshape(bsz, seq_len, HIDDEN),
            pos,
            jnp.transpose(gate_t, (0, 1, 3, 2)))
